# baseline (speedup 1.0000x reference)
import functools

import jax
import jax.numpy as jnp
from jax.experimental import pallas as pl
from jax.experimental.pallas import tpu as pltpu

D_MODEL = 1024
POOL = 4096
K_MIN = 32
K_MAX = 256
TB = 64


def _lane_iota(shape):
    return jax.lax.broadcasted_iota(jnp.int32, shape, dimension=len(shape) - 1)


def _partner(a, s, up):
    n = a.shape[-1]
    ax = a.ndim - 1
    return jnp.where(up, pltpu.roll(a, s, ax), pltpu.roll(a, n - s, ax))


def _cmpex(v, ix, w, s, asc):
    lane = _lane_iota((1, v.shape[-1]))
    up = (lane & s) != 0
    pv = _partner(v, s, up)
    pix = _partner(ix, s, up)
    pw = _partner(w, s, up)
    gt = (v > pv) | ((v == pv) & (ix < pix))
    keep = gt ^ (up ^ asc)
    return (jnp.where(keep, v, pv), jnp.where(keep, ix, pix),
            jnp.where(keep, w, pw))


def _topk_sorted(v, ix, w):
    n = v.shape[-1]
    lane = _lane_iota((1, n))
    for m in (2, 4, 8, 16, 32, 64, 128, 256):
        asc = ((lane // m) & 1) == 1
        s = m // 2
        while s >= 1:
            v, ix, w = _cmpex(v, ix, w, s, asc)
            s //= 2
    width = n
    while width > K_MAX:
        p = width // (2 * K_MAX)
        tb = v.shape[0]
        av = v.reshape(tb, p, 2 * K_MAX)
        aix = ix.reshape(tb, p, 2 * K_MAX)
        aw = w.reshape(tb, p, 2 * K_MAX)
        a_v, b_v = av[:, :, :K_MAX], av[:, :, K_MAX:]
        a_ix, b_ix = aix[:, :, :K_MAX], aix[:, :, K_MAX:]
        a_w, b_w2 = aw[:, :, :K_MAX], aw[:, :, K_MAX:]
        ga = (a_v > b_v) | ((a_v == b_v) & (a_ix < b_ix))
        v = jnp.where(ga, a_v, b_v).reshape(tb, p * K_MAX)
        ix = jnp.where(ga, a_ix, b_ix).reshape(tb, p * K_MAX)
        w = jnp.where(ga, a_w, b_w2).reshape(tb, p * K_MAX)
        width = p * K_MAX
        lane_w = _lane_iota((1, width))
        asc = ((lane_w // K_MAX) & 1) == 1
        s = K_MAX // 2
        while s >= 1:
            v, ix, w = _cmpex(v, ix, w, s, asc)
            s //= 2
    return v, ix, w


def _router_body(x_ref, wsel_ref, bsel_ref, ww_ref, bw_ref, w1_ref, b1_ref,
                 w2_ref, b2_ref, idx_ref, pw_ref, sel_ref, k_ref):
    x = x_ref[...]
    sel = jnp.dot(x, wsel_ref[...]) + bsel_ref[...]
    sel_ref[...] = sel
    w = jnp.dot(x, ww_ref[...]) + bw_ref[...]
    h = jnp.maximum(jnp.dot(x, w1_ref[...]) + b1_ref[...], 0.0)
    c = jax.nn.sigmoid(jnp.dot(h, w2_ref[...].reshape(-1, 1))[:, 0]
                       + b2_ref[0, 0])
    k = (K_MIN + c * (K_MAX - K_MIN)).astype(jnp.int32)
    k_ref[...] = k[None, None, :]
    ix0 = _lane_iota(sel.shape)
    _, ix, pw = _topk_sorted(sel, ix0, w)
    idx_ref[...] = ix
    pos = _lane_iota((1, K_MAX))
    pw_ref[...] = pw * (pos < k[:, None]).astype(jnp.float32)


def _run_router(xf, W_sel, b_sel, W_w, b_w, W1, b1, W2, b2):
    n_tok = xf.shape[0]
    grid = (n_tok // TB,)
    const = lambda *_: (0, 0)
    out_shapes = (
        jax.ShapeDtypeStruct((n_tok, K_MAX), jnp.int32),
        jax.ShapeDtypeStruct((n_tok, K_MAX), jnp.float32),
        jax.ShapeDtypeStruct((n_tok, POOL), jnp.float32),
        jax.ShapeDtypeStruct((n_tok // TB, 1, TB), jnp.int32),
    )
    return pl.pallas_call(
        _router_body,
        grid=grid,
        in_specs=[
            pl.BlockSpec((TB, D_MODEL), lambda i: (i, 0)),
            pl.BlockSpec((D_MODEL, POOL), const),
            pl.BlockSpec((1, POOL), const),
            pl.BlockSpec((D_MODEL, POOL), const),
            pl.BlockSpec((1, POOL), const),
            pl.BlockSpec((D_MODEL, D_MODEL // 4), const),
            pl.BlockSpec((1, D_MODEL // 4), const),
            pl.BlockSpec((1, D_MODEL // 4), const),
            pl.BlockSpec((1, 1), const),
        ],
        out_specs=(
            pl.BlockSpec((TB, K_MAX), lambda i: (i, 0)),
            pl.BlockSpec((TB, K_MAX), lambda i: (i, 0)),
            pl.BlockSpec((TB, POOL), lambda i: (i, 0)),
            pl.BlockSpec((1, 1, TB), lambda i: (i, 0, 0)),
        ),
        out_shape=out_shapes,
        compiler_params=pltpu.CompilerParams(
            dimension_semantics=("parallel",)),
    )(xf, W_sel, b_sel.reshape(1, POOL), W_w, b_w.reshape(1, POOL),
      W1, b1.reshape(1, -1), W2.reshape(1, -1), b2.reshape(1, 1))


def kernel(x, W_sel, b_sel, W_w, b_w, W1, b1, W2, b2):
    batch, seq, _ = x.shape
    xf = x.reshape(batch * seq, D_MODEL)
    idx, pw, sel, kv = _run_router(xf, W_sel, b_sel, W_w, b_w, W1, b1, W2, b2)
    return (idx.reshape(batch, seq, K_MAX),
            pw.reshape(batch, seq, K_MAX),
            sel.reshape(batch, seq, POOL),
            kv.reshape(batch, seq))


# speedup vs baseline: 1.2660x; 1.2660x over previous
import functools

import jax
import jax.numpy as jnp
from jax.experimental import pallas as pl
from jax.experimental.pallas import tpu as pltpu
from jax.experimental.pallas import tpu_sc as plsc

D_MODEL = 1024
POOL = 4096
K_MIN = 32
K_MAX = 256
TB = 128
LANES = 16


def _lane_iota(shape):
    return jax.lax.broadcasted_iota(jnp.int32, shape, dimension=len(shape) - 1)


def _partner(a, s, up):
    n = a.shape[-1]
    ax = a.ndim - 1
    return jnp.where(up, pltpu.roll(a, s, ax), pltpu.roll(a, n - s, ax))


def _cmpex(v, ix, s, asc):
    lane = _lane_iota((1, v.shape[-1]))
    up = (lane & s) != 0
    pv = _partner(v, s, up)
    pix = _partner(ix, s, up)
    gt = (v > pv) | ((v == pv) & (ix < pix))
    keep = gt ^ (up ^ asc)
    return jnp.where(keep, v, pv), jnp.where(keep, ix, pix)


def _topk_sorted(v, ix):
    n = v.shape[-1]
    lane = _lane_iota((1, n))
    for m in (2, 4, 8, 16, 32, 64, 128, 256):
        asc = ((lane // m) & 1) == 1
        s = m // 2
        while s >= 1:
            v, ix = _cmpex(v, ix, s, asc)
            s //= 2
    width = n
    while width > K_MAX:
        p = width // (2 * K_MAX)
        tb = v.shape[0]
        av = v.reshape(tb, p, 2 * K_MAX)
        aix = ix.reshape(tb, p, 2 * K_MAX)
        a_v, b_v = av[:, :, :K_MAX], av[:, :, K_MAX:]
        a_ix, b_ix = aix[:, :, :K_MAX], aix[:, :, K_MAX:]
        ga = (a_v > b_v) | ((a_v == b_v) & (a_ix < b_ix))
        v = jnp.where(ga, a_v, b_v).reshape(tb, p * K_MAX)
        ix = jnp.where(ga, a_ix, b_ix).reshape(tb, p * K_MAX)
        width = p * K_MAX
        lane_w = _lane_iota((1, width))
        asc = ((lane_w // K_MAX) & 1) == 1
        s = K_MAX // 2
        while s >= 1:
            v, ix = _cmpex(v, ix, s, asc)
            s //= 2
    return v, ix


def _router_body(x_ref, wsel_ref, bsel_ref, ww_ref, bw_ref, w1_ref, b1_ref,
                 w2_ref, b2_ref, idx_ref, w_ref, sel_ref, k_ref, mask_ref):
    x = x_ref[...]
    sel = jnp.dot(x, wsel_ref[...]) + bsel_ref[...]
    sel_ref[...] = sel
    w_ref[...] = jnp.dot(x, ww_ref[...]) + bw_ref[...]
    h = jnp.maximum(jnp.dot(x, w1_ref[...]) + b1_ref[...], 0.0)
    c = jax.nn.sigmoid(jnp.dot(h, w2_ref[...].reshape(-1, 1))[:, 0]
                       + b2_ref[0, 0])
    k = (K_MIN + c * (K_MAX - K_MIN)).astype(jnp.int32)
    k_ref[...] = k[None, None, :]
    mask_ref[...] = (_lane_iota((1, K_MAX)) < k[:, None]).astype(jnp.float32)
    _, ix = _topk_sorted(sel, _lane_iota(sel.shape))
    idx_ref[...] = ix


def _run_router(xf, W_sel, b_sel, W_w, b_w, W1, b1, W2, b2):
    n_tok = xf.shape[0]
    grid = (n_tok // TB,)
    const = lambda *_: (0, 0)
    out_shapes = (
        jax.ShapeDtypeStruct((n_tok, K_MAX), jnp.int32),
        jax.ShapeDtypeStruct((n_tok, POOL), jnp.float32),
        jax.ShapeDtypeStruct((n_tok, POOL), jnp.float32),
        jax.ShapeDtypeStruct((n_tok // TB, 1, TB), jnp.int32),
        jax.ShapeDtypeStruct((n_tok, K_MAX), jnp.float32),
    )
    return pl.pallas_call(
        _router_body,
        grid=grid,
        in_specs=[
            pl.BlockSpec((TB, D_MODEL), lambda i: (i, 0)),
            pl.BlockSpec((D_MODEL, POOL), const),
            pl.BlockSpec((1, POOL), const),
            pl.BlockSpec((D_MODEL, POOL), const),
            pl.BlockSpec((1, POOL), const),
            pl.BlockSpec((D_MODEL, D_MODEL // 4), const),
            pl.BlockSpec((1, D_MODEL // 4), const),
            pl.BlockSpec((1, D_MODEL // 4), const),
            pl.BlockSpec((1, 1), const),
        ],
        out_specs=(
            pl.BlockSpec((TB, K_MAX), lambda i: (i, 0)),
            pl.BlockSpec((TB, POOL), lambda i: (i, 0)),
            pl.BlockSpec((TB, POOL), lambda i: (i, 0)),
            pl.BlockSpec((1, 1, TB), lambda i: (i, 0, 0)),
            pl.BlockSpec((TB, K_MAX), lambda i: (i, 0)),
        ),
        out_shape=out_shapes,
        compiler_params=pltpu.CompilerParams(
            dimension_semantics=("parallel",)),
    )(xf, W_sel, b_sel.reshape(1, POOL), W_w, b_w.reshape(1, POOL),
      W1, b1.reshape(1, -1), W2.reshape(1, -1), b2.reshape(1, 1))


def _sc_gather_masked(wsc, idx, mask):
    n_tok = wsc.shape[0]
    num_cores, num_subcores = 2, 16
    nw = num_cores * num_subcores
    tpw = n_tok // nw
    mesh = plsc.VectorSubcoreMesh(core_axis_name="c", subcore_axis_name="s")

    @functools.partial(
        pl.kernel, mesh=mesh,
        out_type=jax.ShapeDtypeStruct((n_tok, K_MAX), jnp.float32),
        compiler_params=pltpu.CompilerParams(needs_layout_passes=False),
        scratch_types=[
            pltpu.VMEM((POOL,), jnp.float32),
            pltpu.VMEM((K_MAX,), jnp.int32),
            pltpu.VMEM((K_MAX,), jnp.float32),
            pltpu.VMEM((K_MAX,), jnp.float32),
        ],
    )
    def run(w_hbm, idx_hbm, m_hbm, out_hbm, wrow_v, irow_v, mrow_v, orow_v):
        wid = jax.lax.axis_index("s") * num_cores + jax.lax.axis_index("c")
        base = wid * tpw

        def body(t, carry):
            tok = base + t
            pltpu.sync_copy(w_hbm.at[tok], wrow_v)
            pltpu.sync_copy(idx_hbm.at[tok], irow_v)
            pltpu.sync_copy(m_hbm.at[tok], mrow_v)
            for j in range(K_MAX // LANES):
                i16 = irow_v[pl.ds(j * LANES, LANES)]
                g = plsc.load_gather(wrow_v, [i16])
                orow_v[pl.ds(j * LANES, LANES)] = g * mrow_v[pl.ds(j * LANES, LANES)]
            pltpu.sync_copy(orow_v, out_hbm.at[tok])
            return carry

        jax.lax.fori_loop(0, tpw, body, 0)

    return run(wsc, idx, mask)


def kernel(x, W_sel, b_sel, W_w, b_w, W1, b1, W2, b2):
    batch, seq, _ = x.shape
    xf = x.reshape(batch * seq, D_MODEL)
    idx, wsc, sel, kv, mask = _run_router(xf, W_sel, b_sel, W_w, b_w,
                                          W1, b1, W2, b2)
    kvf = kv.reshape(batch * seq)
    pw = _sc_gather_masked(wsc, idx, mask)
    return (idx.reshape(batch, seq, K_MAX),
            pw.reshape(batch, seq, K_MAX),
            sel.reshape(batch, seq, POOL),
            kvf.reshape(batch, seq))


# speedup vs baseline: 1.8903x; 1.4932x over previous
import functools

import jax
import jax.numpy as jnp
from jax.experimental import pallas as pl
from jax.experimental.pallas import tpu as pltpu
from jax.experimental.pallas import tpu_sc as plsc

D_MODEL = 1024
POOL = 4096
K_MIN = 32
K_MAX = 256
TB = 128
LANES = 16


def _lane_iota(shape):
    return jax.lax.broadcasted_iota(jnp.int32, shape, dimension=len(shape) - 1)


def _partner(a, s, up):
    n = a.shape[-1]
    ax = a.ndim - 1
    return jnp.where(up, pltpu.roll(a, s, ax), pltpu.roll(a, n - s, ax))


def _cmpex(v, ix, s, asc):
    lane = _lane_iota((1, v.shape[-1]))
    up = (lane & s) != 0
    pv = _partner(v, s, up)
    pix = _partner(ix, s, up)
    keep = (v > pv) ^ (up ^ asc)
    return jnp.where(keep, v, pv), jnp.where(keep, ix, pix)


def _stable_fix(v, ix, o):
    n = v.shape[-1]
    lane = _lane_iota((1, n))
    up = ((lane - o) & 1) == 1
    pv = _partner(v, 1, up)
    pix = _partner(ix, 1, up)
    gt = (v > pv) | ((v == pv) & (ix < pix))
    keep = (gt ^ up) | (lane < o) | (lane >= n - ((n - o) % 2))
    return jnp.where(keep, v, pv), jnp.where(keep, ix, pix)


def _topk_sorted(v, ix):
    n = v.shape[-1]
    lane = _lane_iota((1, n))
    for m in (2, 4, 8, 16, 32, 64, 128, 256):
        asc = ((lane // m) & 1) == 1
        s = m // 2
        while s >= 1:
            v, ix = _cmpex(v, ix, s, asc)
            s //= 2
    width = n
    while width > K_MAX:
        p = width // (2 * K_MAX)
        tb = v.shape[0]
        av = v.reshape(tb, p, 2 * K_MAX)
        aix = ix.reshape(tb, p, 2 * K_MAX)
        a_v, b_v = av[:, :, :K_MAX], av[:, :, K_MAX:]
        a_ix, b_ix = aix[:, :, :K_MAX], aix[:, :, K_MAX:]
        ga = (a_v > b_v) | ((a_v == b_v) & (a_ix < b_ix))
        v = jnp.where(ga, a_v, b_v).reshape(tb, p * K_MAX)
        ix = jnp.where(ga, a_ix, b_ix).reshape(tb, p * K_MAX)
        width = p * K_MAX
        lane_w = _lane_iota((1, width))
        asc = ((lane_w // K_MAX) & 1) == 1
        s = K_MAX // 2
        while s >= 1:
            v, ix = _cmpex(v, ix, s, asc)
            s //= 2
    for o in (0, 1, 0):
        v, ix = _stable_fix(v, ix, o)
    return v, ix


def _router_body(x_ref, wsel_ref, bsel_ref, ww_ref, bw_ref, w1_ref, b1_ref,
                 w2_ref, b2_ref, idx_ref, w_ref, sel_ref, k_ref, mask_ref):
    x = x_ref[...]
    sel = jnp.dot(x, wsel_ref[...]) + bsel_ref[...]
    sel_ref[...] = sel
    w_ref[...] = jnp.dot(x, ww_ref[...]) + bw_ref[...]
    h = jnp.maximum(jnp.dot(x, w1_ref[...]) + b1_ref[...], 0.0)
    c = jax.nn.sigmoid(jnp.dot(h, w2_ref[...].reshape(-1, 1))[:, 0]
                       + b2_ref[0, 0])
    k = (K_MIN + c * (K_MAX - K_MIN)).astype(jnp.int32)
    k_ref[...] = k[None, None, :]
    mask_ref[...] = (_lane_iota((1, K_MAX)) < k[:, None]).astype(jnp.float32)
    _, ix = _topk_sorted(sel, _lane_iota(sel.shape).astype(jnp.int16))
    idx_ref[...] = ix.astype(jnp.int32)


def _run_router(xf, W_sel, b_sel, W_w, b_w, W1, b1, W2, b2):
    n_tok = xf.shape[0]
    grid = (n_tok // TB,)
    const = lambda *_: (0, 0)
    out_shapes = (
        jax.ShapeDtypeStruct((n_tok, K_MAX), jnp.int32),
        jax.ShapeDtypeStruct((n_tok, POOL), jnp.float32),
        jax.ShapeDtypeStruct((n_tok, POOL), jnp.float32),
        jax.ShapeDtypeStruct((n_tok // TB, 1, TB), jnp.int32),
        jax.ShapeDtypeStruct((n_tok, K_MAX), jnp.float32),
    )
    return pl.pallas_call(
        _router_body,
        grid=grid,
        in_specs=[
            pl.BlockSpec((TB, D_MODEL), lambda i: (i, 0)),
            pl.BlockSpec((D_MODEL, POOL), const),
            pl.BlockSpec((1, POOL), const),
            pl.BlockSpec((D_MODEL, POOL), const),
            pl.BlockSpec((1, POOL), const),
            pl.BlockSpec((D_MODEL, D_MODEL // 4), const),
            pl.BlockSpec((1, D_MODEL // 4), const),
            pl.BlockSpec((1, D_MODEL // 4), const),
            pl.BlockSpec((1, 1), const),
        ],
        out_specs=(
            pl.BlockSpec((TB, K_MAX), lambda i: (i, 0)),
            pl.BlockSpec((TB, POOL), lambda i: (i, 0)),
            pl.BlockSpec((TB, POOL), lambda i: (i, 0)),
            pl.BlockSpec((1, 1, TB), lambda i: (i, 0, 0)),
            pl.BlockSpec((TB, K_MAX), lambda i: (i, 0)),
        ),
        out_shape=out_shapes,
        compiler_params=pltpu.CompilerParams(
            dimension_semantics=("parallel",)),
    )(xf, W_sel, b_sel.reshape(1, POOL), W_w, b_w.reshape(1, POOL),
      W1, b1.reshape(1, -1), W2.reshape(1, -1), b2.reshape(1, 1))


def _sc_gather_masked(wsc, idx, mask):
    n_tok = wsc.shape[0]
    num_cores, num_subcores = 2, 16
    nw = num_cores * num_subcores
    tpw = n_tok // nw
    mesh = plsc.VectorSubcoreMesh(core_axis_name="c", subcore_axis_name="s")

    @functools.partial(
        pl.kernel, mesh=mesh,
        out_type=jax.ShapeDtypeStruct((n_tok, K_MAX), jnp.float32),
        compiler_params=pltpu.CompilerParams(needs_layout_passes=False),
        scratch_types=[
            pltpu.VMEM((POOL,), jnp.float32),
            pltpu.VMEM((K_MAX,), jnp.int32),
            pltpu.VMEM((K_MAX,), jnp.float32),
            pltpu.VMEM((K_MAX,), jnp.float32),
        ],
    )
    def run(w_hbm, idx_hbm, m_hbm, out_hbm, wrow_v, irow_v, mrow_v, orow_v):
        wid = jax.lax.axis_index("s") * num_cores + jax.lax.axis_index("c")
        base = wid * tpw

        def body(t, carry):
            tok = base + t
            pltpu.sync_copy(w_hbm.at[tok], wrow_v)
            pltpu.sync_copy(idx_hbm.at[tok], irow_v)
            pltpu.sync_copy(m_hbm.at[tok], mrow_v)
            for j in range(K_MAX // LANES):
                i16 = irow_v[pl.ds(j * LANES, LANES)]
                g = plsc.load_gather(wrow_v, [i16])
                orow_v[pl.ds(j * LANES, LANES)] = g * mrow_v[pl.ds(j * LANES, LANES)]
            pltpu.sync_copy(orow_v, out_hbm.at[tok])
            return carry

        jax.lax.fori_loop(0, tpw, body, 0)

    return run(wsc, idx, mask)


def kernel(x, W_sel, b_sel, W_w, b_w, W1, b1, W2, b2):
    batch, seq, _ = x.shape
    xf = x.reshape(batch * seq, D_MODEL)
    idx, wsc, sel, kv, mask = _run_router(xf, W_sel, b_sel, W_w, b_w,
                                          W1, b1, W2, b2)
    kvf = kv.reshape(batch * seq)
    pw = _sc_gather_masked(wsc, idx, mask)
    return (idx.reshape(batch, seq, K_MAX),
            pw.reshape(batch, seq, K_MAX),
            sel.reshape(batch, seq, POOL),
            kvf.reshape(batch, seq))
